# Initial kernel scaffold; baseline (speedup 1.0000x reference)
import jax
import jax.numpy as jnp
from jax.experimental import pallas as pl


def kernel(exyz):
    raise NotImplementedError("write your pallas kernel here")


# speedup vs baseline: 10.7206x; 10.7206x over previous
import functools

import jax
import jax.numpy as jnp
from jax import lax
from jax.experimental import pallas as pl
from jax.experimental.pallas import tpu as pltpu
from jax.experimental.pallas import tpu_sc as plsc

N = 32768
G = 1024
K = 32
R, C = 256, 128
QB = 128


def _fps_body(x_ref, y_ref, z_ref, cidx_ref, cx_ref, cy_ref, cz_ref):
    x = x_ref[...]
    y = y_ref[...]
    z = z_ref[...]
    flat = (lax.broadcasted_iota(jnp.int32, (R, C), 0) * C
            + lax.broadcasted_iota(jnp.int32, (R, C), 1))
    gflat = (lax.broadcasted_iota(jnp.int32, (8, 128), 0) * 128
             + lax.broadcasted_iota(jnp.int32, (8, 128), 1))

    def extract(last):
        sel = flat == last
        lx = jnp.sum(jnp.where(sel, x, 0.0))
        ly = jnp.sum(jnp.where(sel, y, 0.0))
        lz = jnp.sum(jnp.where(sel, z, 0.0))
        return lx, ly, lz

    def step(i, carry):
        dists, last, acc_i, acc_x, acc_y, acc_z = carry
        lx, ly, lz = extract(last)
        prev_sel = gflat == (i - 1)
        acc_x = jnp.where(prev_sel, lx, acc_x)
        acc_y = jnp.where(prev_sel, ly, acc_y)
        acc_z = jnp.where(prev_sel, lz, acc_z)
        dx = x - lx
        dy = y - ly
        dz = z - lz
        d = dx * dx + dy * dy + dz * dz
        dists = jnp.minimum(dists, d)
        m = jnp.max(dists)
        nxt = jnp.min(jnp.where(dists == m, flat, N))
        acc_i = jnp.where(gflat == i, nxt, acc_i)
        return dists, nxt, acc_i, acc_x, acc_y, acc_z

    dists0 = jnp.full((R, C), jnp.inf, dtype=jnp.float32)
    zf = jnp.zeros((8, 128), jnp.float32)
    carry = (dists0, jnp.int32(0), jnp.zeros((8, 128), jnp.int32), zf, zf, zf)
    _, last, acc_i, acc_x, acc_y, acc_z = lax.fori_loop(1, G, step, carry)
    lx, ly, lz = extract(last)
    lsel = gflat == (G - 1)
    cidx_ref[...] = acc_i
    cx_ref[...] = jnp.where(lsel, lx, acc_x)
    cy_ref[...] = jnp.where(lsel, ly, acc_y)
    cz_ref[...] = jnp.where(lsel, lz, acc_z)


_fps_call = pl.pallas_call(
    _fps_body,
    out_shape=[
        jax.ShapeDtypeStruct((8, 128), jnp.int32),
        jax.ShapeDtypeStruct((8, 128), jnp.float32),
        jax.ShapeDtypeStruct((8, 128), jnp.float32),
        jax.ShapeDtypeStruct((8, 128), jnp.float32),
    ],
)


def _knn_body(qmat_ref, pmat_ref, idx_ref):
    qmat = qmat_ref[...]
    pmat = pmat_ref[...]
    qx = qmat[:, 0:1]
    qy = qmat[:, 1:2]
    qz = qmat[:, 2:3]
    px = pmat[0:1, :]
    py = pmat[1:2, :]
    pz = pmat[2:3, :]
    q2 = qx * qx + qy * qy + qz * qz
    p2 = px * px + py * py + pz * pz
    qp = lax.dot_general(qmat, pmat,
                         dimension_numbers=(((1,), (0,)), ((), ())))
    d = q2 - 2.0 * qp + p2
    niota = lax.broadcasted_iota(jnp.int32, (QB, N), 1)
    kio = lax.broadcasted_iota(jnp.int32, (QB, K), 1)

    def sel_step(k, carry):
        d, acc = carry
        m = jnp.min(d, axis=1, keepdims=True)
        idx = jnp.min(jnp.where(d == m, niota, N), axis=1, keepdims=True)
        acc = jnp.where(kio == k, idx, acc)
        d = jnp.where(niota == idx, jnp.float32(jnp.inf), d)
        return d, acc

    _, acc = lax.fori_loop(0, K, sel_step,
                           (d, jnp.zeros((QB, K), jnp.int32)))
    idx_ref[...] = acc


_knn_call = pl.pallas_call(
    _knn_body,
    grid=(G // QB,),
    in_specs=[
        pl.BlockSpec((QB, 8), lambda i: (i, 0)),
        pl.BlockSpec((8, N), lambda i: (0, 0)),
    ],
    out_specs=pl.BlockSpec((QB, K), lambda i: (i, 0)),
    out_shape=jax.ShapeDtypeStruct((G, K), jnp.int32),
)


_NC, _NS = 2, 16
_NW = _NC * _NS
_NROWS = G * K + G
_BPW = _NROWS // _NW
_DPAD = 16


def _gather_body(table_hbm, idx_hbm, out_hbm, idx_v, rows_v, sem):
    wid = lax.axis_index("s") * _NC + lax.axis_index("c")
    base = wid * _BPW
    pltpu.sync_copy(idx_hbm.at[pl.ds(base, _BPW)], idx_v)
    pltpu.async_copy(table_hbm.at[idx_v], rows_v, sem).wait()
    pltpu.sync_copy(rows_v, out_hbm.at[pl.ds(base, _BPW)])


@functools.cache
def _gather_call():
    return functools.partial(
        pl.kernel,
        mesh=plsc.VectorSubcoreMesh(core_axis_name="c", subcore_axis_name="s"),
        compiler_params=pltpu.CompilerParams(use_tc_tiling_on_sc=False),
        out_type=jax.ShapeDtypeStruct((_NROWS, _DPAD), jnp.float32),
        scratch_types=[
            pltpu.VMEM((_BPW,), jnp.int32),
            pltpu.VMEM((_BPW, _DPAD), jnp.float32),
            pltpu.SemaphoreType.DMA,
        ],
    )(_gather_body)


def kernel(exyz):
    e = exyz[0]
    x = e[:, 1].reshape(R, C)
    y = e[:, 2].reshape(R, C)
    z = e[:, 3].reshape(R, C)
    cidx8, cx8, cy8, cz8 = _fps_call(x, y, z)
    cidx = cidx8.reshape(G)
    qmat = jnp.concatenate(
        [cx8.reshape(G, 1), cy8.reshape(G, 1), cz8.reshape(G, 1),
         jnp.zeros((G, 5), jnp.float32)], axis=1)
    pmat = jnp.concatenate(
        [e[:, 1].reshape(1, N), e[:, 2].reshape(1, N), e[:, 3].reshape(1, N),
         jnp.zeros((5, N), jnp.float32)], axis=0)
    idx = _knn_call(qmat, pmat)
    table = jnp.pad(e, ((0, 0), (0, _DPAD - 4)))
    all_idx = jnp.concatenate([idx.reshape(-1), cidx])
    rows = _gather_call()(table, all_idx)
    neighborhood = rows[: G * K, :4].reshape(1, G, K, 4)
    centers_e = rows[G * K :, :4].reshape(1, G, 4)
    return neighborhood, centers_e
